# initial kernel scaffold (unmeasured)
import jax
import jax.numpy as jnp
from jax import lax
from jax.experimental import pallas as pl
from jax.experimental.pallas import tpu as pltpu

N_DEV = 8


def kernel(x, w_mat):
    M, k_shard = x.shape
    K, N = w_mat.shape
    m_out = M // N_DEV

    def body(x_ref, w_ref, out_ref, xall, amax_src, amax_all,
             send_sems, recv_sems, amax_send_sems, amax_recv_sems):
        my = lax.axis_index("i")

        data_rdmas = []
        for k in range(1, N_DEV):
            d = (my + k) % N_DEV
            rdma = pltpu.make_async_remote_copy(
                src_ref=x_ref.at[pl.ds(d * m_out, m_out), :],
                dst_ref=xall.at[my],
                send_sem=send_sems.at[k],
                recv_sem=recv_sems.at[my],
                device_id=(d,),
                device_id_type=pl.DeviceIdType.MESH,
            )
            rdma.start()
            data_rdmas.append(rdma)

        own = x_ref[pl.ds(my * m_out, m_out), :]
        out_ref[...] = jnp.dot(
            own, w_ref[pl.ds(my * k_shard, k_shard), :],
            preferred_element_type=jnp.float32,
        )
        for k in range(1, N_DEV):
            j = (my + k) % N_DEV
            recv = pltpu.make_async_remote_copy(
                src_ref=xall.at[j],
                dst_ref=xall.at[j],
                send_sem=send_sems.at[k],
                recv_sem=recv_sems.at[j],
                device_id=(j,),
                device_id_type=pl.DeviceIdType.MESH,
            )
            recv.wait_recv()
            out_ref[...] += jnp.dot(
                xall[j], w_ref[pl.ds(j * k_shard, k_shard), :],
                preferred_element_type=jnp.float32,
            )

        for rdma in data_rdmas:
            rdma.wait_send()

        local_amax = jnp.max(jnp.abs(out_ref[...]))
        amax_src[...] = jnp.full((1, 128), local_amax, dtype=jnp.float32)
        amax_rdmas = []
        for k in range(1, N_DEV):
            d = (my + k) % N_DEV
            rdma = pltpu.make_async_remote_copy(
                src_ref=amax_src,
                dst_ref=amax_all.at[my],
                send_sem=amax_send_sems.at[k],
                recv_sem=amax_recv_sems.at[my],
                device_id=(d,),
                device_id_type=pl.DeviceIdType.MESH,
            )
            rdma.start()
            amax_rdmas.append(rdma)
        for k in range(1, N_DEV):
            j = (my + k) % N_DEV
            recv = pltpu.make_async_remote_copy(
                src_ref=amax_src,
                dst_ref=amax_all.at[j],
                send_sem=amax_send_sems.at[k],
                recv_sem=amax_recv_sems.at[j],
                device_id=(j,),
                device_id_type=pl.DeviceIdType.MESH,
            )
            recv.wait_recv()
        for rdma in amax_rdmas:
            rdma.wait_send()

        g_amax = jnp.maximum(jnp.max(amax_all[...]), local_amax)

        scale = g_amax / 448.0
        y = out_ref[...]
        q = jnp.clip(y / scale, -448.0, 448.0).astype(jnp.float8_e4m3fn)
        out_ref[...] = q.astype(jnp.float32) * scale

    return pl.pallas_call(
        body,
        out_shape=jax.ShapeDtypeStruct((m_out, N), jnp.float32),
        in_specs=[
            pl.BlockSpec(memory_space=pltpu.VMEM),
            pl.BlockSpec(memory_space=pltpu.VMEM),
        ],
        out_specs=pl.BlockSpec(memory_space=pltpu.VMEM),
        scratch_shapes=[
            pltpu.VMEM((N_DEV, m_out, k_shard), jnp.bfloat16),
            pltpu.VMEM((1, 128), jnp.float32),
            pltpu.VMEM((N_DEV, 128), jnp.float32),
            pltpu.SemaphoreType.DMA((N_DEV,)),
            pltpu.SemaphoreType.DMA((N_DEV,)),
            pltpu.SemaphoreType.DMA((N_DEV,)),
            pltpu.SemaphoreType.DMA((N_DEV,)),
        ],
    )(x, w_mat)


# baseline (device time: 121894 ns/iter reference)
import jax
import jax.numpy as jnp
from jax import lax
from jax.experimental import pallas as pl
from jax.experimental.pallas import tpu as pltpu

N_DEV = 8
N_HALVES = 4


def kernel(x, w_mat):
    M, k_shard = x.shape
    K, N = w_mat.shape
    m_out = M // N_DEV
    n_half = N // N_HALVES
    n_chunks = N_DEV * N_HALVES

    def body(x_ref, w_hbm, out_ref, xbf, xall, wbuf, amax_src, amax_all,
             send_sems, recv_sems, amax_send_sems, amax_recv_sems, w_sems):
        my = lax.axis_index("i")

        barrier_sem = pltpu.get_barrier_semaphore()
        for k in range(1, N_DEV):
            pl.semaphore_signal(
                barrier_sem, inc=1,
                device_id=((my + k) % N_DEV,),
                device_id_type=pl.DeviceIdType.MESH,
            )
        pl.semaphore_wait(barrier_sem, N_DEV - 1)

        def w_chunk_copy(c):
            kk, h = c // N_HALVES, c % N_HALVES
            j = (my + kk) % N_DEV
            return pltpu.make_async_copy(
                w_hbm.at[pl.ds(j * k_shard, k_shard),
                         pl.ds(h * n_half, n_half)],
                wbuf.at[c % 2], w_sems.at[c % 2],
            )

        w_copies = {}
        for c in range(2):
            w_copies[c] = w_chunk_copy(c)
            w_copies[c].start()

        amax_all[...] = jnp.zeros_like(amax_all)

        xbf[...] = x_ref[...].astype(jnp.bfloat16)

        data_rdmas = []
        for k in range(1, N_DEV):
            d = (my + k) % N_DEV
            rdma = pltpu.make_async_remote_copy(
                src_ref=xbf.at[pl.ds(d * m_out, m_out), :],
                dst_ref=xall.at[my],
                send_sem=send_sems.at[k],
                recv_sem=recv_sems.at[my],
                device_id=(d,),
                device_id_type=pl.DeviceIdType.MESH,
            )
            rdma.start()
            data_rdmas.append(rdma)

        import os
        serial_recv = os.environ.get("OVERLAP_RECV") != "1"
        if serial_recv:
            for k in range(1, N_DEV):
                j = (my + k) % N_DEV
                pltpu.make_async_remote_copy(
                    src_ref=xall.at[j],
                    dst_ref=xall.at[j],
                    send_sem=send_sems.at[k],
                    recv_sem=recv_sems.at[j],
                    device_id=(j,),
                    device_id_type=pl.DeviceIdType.MESH,
                ).wait_recv()

        for c in range(n_chunks):
            k, h = c // N_HALVES, c % N_HALVES
            j = (my + k) % N_DEV
            if k == 0:
                a = xbf[pl.ds(my * m_out, m_out), :]
            else:
                a = xall[j]
                if h == 0 and not serial_recv:
                    recv = pltpu.make_async_remote_copy(
                        src_ref=xall.at[j],
                        dst_ref=xall.at[j],
                        send_sem=send_sems.at[k],
                        recv_sem=recv_sems.at[j],
                        device_id=(j,),
                        device_id_type=pl.DeviceIdType.MESH,
                    )
                    recv.wait_recv()
            w_copies[c].wait()
            contrib = jnp.dot(
                a, wbuf[c % 2].astype(jnp.bfloat16),
                preferred_element_type=jnp.float32,
            )
            nsl = pl.ds(h * n_half, n_half)
            if k == 0:
                out_ref[:, nsl] = contrib
            else:
                out_ref[:, nsl] += contrib
            if c + 2 < n_chunks:
                w_copies[c + 2] = w_chunk_copy(c + 2)
                w_copies[c + 2].start()

        for rdma in data_rdmas:
            rdma.wait_send()

        local_amax = jnp.float32(0.0)
        for c in range(N_DEV):
            sl = pl.ds(c * (N // N_DEV), N // N_DEV)
            local_amax = jnp.maximum(
                local_amax, jnp.max(jnp.abs(out_ref[:, sl]))
            )
        amax_src[...] = jnp.full((8, 128), local_amax, dtype=jnp.float32)
        amax_rdmas = []
        for k in range(1, N_DEV):
            d = (my + k) % N_DEV
            rdma = pltpu.make_async_remote_copy(
                src_ref=amax_src,
                dst_ref=amax_all.at[my],
                send_sem=amax_send_sems.at[k],
                recv_sem=amax_recv_sems.at[my],
                device_id=(d,),
                device_id_type=pl.DeviceIdType.MESH,
            )
            rdma.start()
            amax_rdmas.append(rdma)
        for k in range(1, N_DEV):
            j = (my + k) % N_DEV
            recv = pltpu.make_async_remote_copy(
                src_ref=amax_src,
                dst_ref=amax_all.at[j],
                send_sem=amax_send_sems.at[k],
                recv_sem=amax_recv_sems.at[j],
                device_id=(j,),
                device_id_type=pl.DeviceIdType.MESH,
            )
            recv.wait_recv()
        for rdma in amax_rdmas:
            rdma.wait_send()

        g_amax = jnp.maximum(jnp.max(amax_all[...]), local_amax)

        import os
        if os.environ.get("SKIP_EPILOGUE") == "1":
            return
        scale = g_amax / 448.0
        for c in range(N_DEV):
            sl = pl.ds(c * (N // N_DEV), N // N_DEV)
            y = out_ref[:, sl]
            q = jnp.clip(y / scale, -448.0, 448.0).astype(jnp.float8_e4m3fn)
            out_ref[:, sl] = q.astype(jnp.float32) * scale

    return pl.pallas_call(
        body,
        out_shape=jax.ShapeDtypeStruct((m_out, N), jnp.float32),
        in_specs=[
            pl.BlockSpec(memory_space=pltpu.VMEM),
            pl.BlockSpec(memory_space=pl.ANY),
        ],
        out_specs=pl.BlockSpec(memory_space=pltpu.VMEM),
        scratch_shapes=[
            pltpu.VMEM((M, k_shard), jnp.bfloat16),
            pltpu.VMEM((N_DEV, m_out, k_shard), jnp.bfloat16),
            pltpu.VMEM((2, k_shard, n_half), jnp.float32),
            pltpu.VMEM((8, 128), jnp.float32),
            pltpu.VMEM((N_DEV, 8, 128), jnp.float32),
            pltpu.SemaphoreType.DMA((N_DEV,)),
            pltpu.SemaphoreType.DMA((N_DEV,)),
            pltpu.SemaphoreType.DMA((N_DEV,)),
            pltpu.SemaphoreType.DMA((N_DEV,)),
            pltpu.SemaphoreType.DMA((2,)),
        ],
        compiler_params=pltpu.CompilerParams(
            vmem_limit_bytes=40 * 1024 * 1024,
            collective_id=0,
        ),
    )(x, w_mat)
